# baseline (speedup 1.0000x reference)
import functools

import jax
import jax.numpy as jnp
from jax import lax
from jax.experimental import pallas as pl
from jax.experimental.pallas import tpu as pltpu
from jax.experimental.pallas import tpu_sc as plsc

NC = 2
NS = 16
L = 16
NW = NC * NS

IDX_CHUNK = 128


def _make_bpr(B, D):
    bpw = B // NW
    n_seg = bpw // IDX_CHUNK
    mesh = plsc.VectorSubcoreMesh(core_axis_name="c", subcore_axis_name="s")

    @functools.partial(
        pl.kernel,
        out_type=jax.ShapeDtypeStruct((NW * L,), jnp.float32),
        mesh=mesh,
        compiler_params=pltpu.CompilerParams(needs_layout_passes=False,
                                             use_tc_tiling_on_sc=False),
        scratch_types=[
            pltpu.VMEM((bpw,), jnp.int32),
            pltpu.VMEM((bpw,), jnp.int32),
            pltpu.VMEM((bpw,), jnp.float32),
            pltpu.VMEM((D, bpw), jnp.float32),
            pltpu.VMEM((D, bpw), jnp.float32),
            pltpu.VMEM((L,), jnp.float32),
            pltpu.SemaphoreType.DMA,
        ],
    )
    def bpr(users_hbm, items_hbm, scores_hbm, utT_hbm, itT_hbm, out_hbm,
            uidx, iidx, sc_v, ubuf, ibuf, acc_v, sem):
        cid = lax.axis_index("c")
        sid = lax.axis_index("s")
        wid = sid * NC + cid
        base = wid * bpw

        pltpu.sync_copy(users_hbm.at[pl.ds(base, bpw)], uidx)
        pltpu.sync_copy(items_hbm.at[pl.ds(base, bpw)], iidx)
        pltpu.sync_copy(scores_hbm.at[pl.ds(base, bpw)], sc_v)

        def fetch_seg(c, carry):
            seg = pl.ds(c * L, L)
            u16 = uidx[seg]
            i16 = iidx[seg]
            copies = []
            for d in range(D):
                copies.append(pltpu.async_copy(
                    utT_hbm.at[d].at[u16], ubuf.at[d, seg], sem))
                copies.append(pltpu.async_copy(
                    itT_hbm.at[d].at[i16], ibuf.at[d, seg], sem))
            for cp in copies:
                cp.wait()
            return carry

        lax.fori_loop(0, bpw // L, fetch_seg, jnp.int32(0))

        def chunk_body(c, acc):
            bs = pl.ds(c * L, L)
            pred = jnp.zeros((L,), jnp.float32)
            for d in range(D):
                pred = pred + ubuf[d, bs] * ibuf[d, bs]
            diff = pred - sc_v[bs]
            return acc + diff * diff

        acc = lax.fori_loop(0, bpw // L, chunk_body,
                            jnp.zeros((L,), jnp.float32))
        total = jnp.sum(acc)
        lane = lax.iota(jnp.int32, L)
        acc_v[...] = jnp.where(lane == 0, total, jnp.float32(0.0))
        pltpu.sync_copy(acc_v, out_hbm.at[pl.ds(wid * L, L)])

    return bpr


def kernel(users, items, scores, user_table, item_table):
    B = users.shape[0]
    D = user_table.shape[1]
    bpr = _make_bpr(B, D)
    partials = bpr(users.astype(jnp.int32), items.astype(jnp.int32),
                   scores, user_table.T, item_table.T)
    return jnp.sum(partials) / B


# speedup vs baseline: 5.7176x; 5.7176x over previous
import functools

import jax
import jax.numpy as jnp
from jax import lax
from jax.experimental import pallas as pl
from jax.experimental.pallas import tpu as pltpu
from jax.experimental.pallas import tpu_sc as plsc

NC = 2
NS = 16
L = 16
NW = NC * NS
SEG = 128
OUTW = 1024


def _make_bpr(B, D):
    bpw = B // NW
    n_seg = bpw // SEG
    rpl = 128 // D
    mesh = plsc.VectorSubcoreMesh(core_axis_name="c", subcore_axis_name="s")

    @functools.partial(
        pl.kernel,
        out_type=jax.ShapeDtypeStruct((NW * OUTW,), jnp.float32),
        mesh=mesh,
        compiler_params=pltpu.CompilerParams(needs_layout_passes=False),
        scratch_types=[
            pltpu.VMEM((bpw,), jnp.int32),
            pltpu.VMEM((bpw,), jnp.int32),
            pltpu.VMEM((bpw,), jnp.int32),
            pltpu.VMEM((bpw,), jnp.int32),
            pltpu.VMEM((bpw,), jnp.float32),
            pltpu.VMEM((2, SEG, 128), jnp.float32),
            pltpu.VMEM((2, SEG, 128), jnp.float32),
            pltpu.VMEM((OUTW,), jnp.float32),
            pltpu.SemaphoreType.DMA,
            pltpu.SemaphoreType.DMA,
        ],
    )
    def bpr(users_hbm, items_hbm, scores_hbm, uq_hbm, iq_hbm, out_hbm,
            uidx, iidx, ulin, ilin, sc_v, ubuf, ibuf, acc_v, sem0, sem1):
        cid = lax.axis_index("c")
        sid = lax.axis_index("s")
        wid = sid * NC + cid
        base = wid * bpw

        pltpu.sync_copy(users_hbm.at[pl.ds(base, bpw)], uidx)
        pltpu.sync_copy(items_hbm.at[pl.ds(base, bpw)], iidx)
        pltpu.sync_copy(scores_hbm.at[pl.ds(base, bpw)], sc_v)

        def line_body(c, carry):
            off = pl.ds(c * L, L)
            ulin[off] = lax.shift_right_logical(uidx[off], 2)
            ilin[off] = lax.shift_right_logical(iidx[off], 2)
            return carry

        lax.fori_loop(0, bpw // L, line_body, jnp.int32(0))

        sems = (sem0, sem1)

        def fire(j):
            seg = pl.ds(j * SEG, SEG)
            slot = j % 2
            return (pltpu.async_copy(uq_hbm.at[ulin.at[seg]],
                                     ubuf.at[slot], sems[slot]),
                    pltpu.async_copy(iq_hbm.at[ilin.at[seg]],
                                     ibuf.at[slot], sems[slot]))

        def seg_compute(j, acc):
            slot = j % 2

            def chunk_body(c, acc):
                r0 = c * L
                scv = sc_v[pl.ds(j * SEG + r0, L)]
                u16 = uidx[pl.ds(j * SEG + r0, L)]
                i16 = iidx[pl.ds(j * SEG + r0, L)]
                usel = (u16 & (rpl - 1)) * D
                isel = (i16 & (rpl - 1)) * D
                for k in range(L):
                    r = r0 + k
                    uo = usel[k]
                    io = isel[k]
                    prod = jnp.zeros((L,), jnp.float32)
                    for d0 in range(0, D, L):
                        prod = prod + (ubuf[slot, r, pl.ds(uo + d0, L)] *
                                       ibuf[slot, r, pl.ds(io + d0, L)])
                    diff = jnp.sum(prod) - scv[k]
                    acc = acc + diff * diff
                return acc

            return lax.fori_loop(0, SEG // L, chunk_body, acc)

        acc = jnp.float32(0.0)
        inflight = fire(0)
        for j in range(n_seg):
            nxt = fire(j + 1) if j + 1 < n_seg else ()
            for cp in inflight:
                cp.wait()
            acc = seg_compute(j, acc)
            inflight = nxt
        lane = lax.iota(jnp.int32, L)
        zero = jnp.zeros((L,), jnp.float32)
        acc_v[pl.ds(0, L)] = jnp.where(lane == 0, acc, jnp.float32(0.0))
        for g in range(1, OUTW // L):
            acc_v[pl.ds(g * L, L)] = zero
        pltpu.sync_copy(acc_v, out_hbm.at[pl.ds(wid * OUTW, OUTW)])

    return bpr


def kernel(users, items, scores, user_table, item_table):
    B = users.shape[0]
    V, D = user_table.shape
    lines = V * D // 128
    uq = jnp.reshape(user_table, (lines, 128))
    iq = jnp.reshape(item_table, (lines, 128))
    bpr = _make_bpr(B, D)
    partials = bpr(users.astype(jnp.int32), items.astype(jnp.int32),
                   scores, uq, iq)
    return jnp.sum(partials) / B


# speedup vs baseline: 22.1019x; 3.8656x over previous
import functools

import jax
import jax.numpy as jnp
from jax import lax
from jax.experimental import pallas as pl
from jax.experimental.pallas import tpu as pltpu
from jax.experimental.pallas import tpu_sc as plsc

NC = 2
NS = 16
L = 16
NW = NC * NS
LANES = 128
DEPTH = 8
OUTW = 1024


def _make_bpr(B, D):
    bpw = B // NW
    n_chunks = bpw // L
    mesh = plsc.VectorSubcoreMesh(core_axis_name="c", subcore_axis_name="s")

    @functools.partial(
        pl.kernel,
        out_type=jax.ShapeDtypeStruct((NW * OUTW,), jnp.float32),
        mesh=mesh,
        compiler_params=pltpu.CompilerParams(needs_layout_passes=False),
        scratch_types=[
            pltpu.VMEM((2 * bpw,), jnp.int32),
            pltpu.VMEM((2 * bpw,), jnp.int32),
            pltpu.VMEM((2 * bpw,), jnp.float32),
            pltpu.VMEM((DEPTH, D, LANES), jnp.float32),
            pltpu.VMEM((DEPTH, D, LANES), jnp.float32),
            pltpu.VMEM((OUTW,), jnp.float32),
        ] + [pltpu.SemaphoreType.DMA] * (2 * DEPTH),
    )
    def bpr(users_hbm, items_hbm, scores_hbm, utT_hbm, itT_hbm, out_hbm,
            uidx, iidx, sc_v, ublk, iblk, acc_v, *sems):
        cid = lax.axis_index("c")
        sid = lax.axis_index("s")
        wid = sid * NC + cid
        blk = (wid // 2) * (2 * bpw)
        loc = (wid % 2) * bpw
        pltpu.sync_copy(users_hbm.at[pl.ds(blk, 2 * bpw)], uidx)
        pltpu.sync_copy(items_hbm.at[pl.ds(blk, 2 * bpw)], iidx)
        pltpu.sync_copy(scores_hbm.at[pl.ds(blk, 2 * bpw)], sc_v)

        usem = sems[:DEPTH]
        isem = sems[DEPTH:]
        lane = lax.iota(jnp.int32, L)

        def fire(r, slot, uval, ival):
            uq = pl.multiple_of(
                lax.shift_right_logical(uval, 7) * LANES, LANES)
            iq = pl.multiple_of(
                lax.shift_right_logical(ival, 7) * LANES, LANES)
            pltpu.async_copy(
                utT_hbm.at[:, pl.ds(uq, LANES)], ublk.at[slot], usem[slot])
            pltpu.async_copy(
                itT_hbm.at[:, pl.ds(iq, LANES)], iblk.at[slot], isem[slot])

        def drain(slot):
            pltpu.make_async_copy(
                utT_hbm.at[:, pl.ds(0, LANES)], ublk.at[slot],
                usem[slot]).wait()
            pltpu.make_async_copy(
                itT_hbm.at[:, pl.ds(0, LANES)], iblk.at[slot],
                isem[slot]).wait()

        def compute(slot, uval, ival, sval, acc):
            ul = jnp.full((L,), uval & (LANES - 1), jnp.int32)
            il = jnp.full((L,), ival & (LANES - 1), jnp.int32)
            prod = jnp.zeros((L,), jnp.float32)
            for h in range(D // L):
                rows = h * L + lane
                gu = plsc.load_gather(ublk.at[slot], [rows, ul])
                gi = plsc.load_gather(iblk.at[slot], [rows, il])
                prod = prod + gu * gi
            diff = jnp.sum(prod) - sval
            return acc + diff * diff

        u0 = uidx[pl.ds(loc, L)]
        i0 = iidx[pl.ds(loc, L)]
        for k in range(DEPTH):
            fire(k, k % DEPTH, u0[k], i0[k])

        def step(c, acc):
            off = loc + c * L
            offn = jnp.minimum(off + L, 2 * bpw - L)
            u16 = uidx[pl.ds(off, L)]
            i16 = iidx[pl.ds(off, L)]
            u16n = uidx[pl.ds(offn, L)]
            i16n = iidx[pl.ds(offn, L)]
            scv = sc_v[pl.ds(off, L)]
            for k in range(L):
                r = c * L + k
                ka = k + DEPTH
                if ka < L:
                    ua, ia = u16[ka], i16[ka]
                else:
                    ua, ia = u16n[ka - L], i16n[ka - L]
                slot = k % DEPTH
                drain(slot)
                acc = compute(slot, u16[k], i16[k], scv[k], acc)

                @pl.when(r + DEPTH < bpw)
                def _():
                    fire(r + DEPTH, slot, ua, ia)
            return acc

        acc = lax.fori_loop(0, n_chunks, step, jnp.float32(0.0))

        zero = jnp.zeros((L,), jnp.float32)
        acc_v[pl.ds(0, L)] = jnp.where(lane == 0, acc, jnp.float32(0.0))
        for g in range(1, OUTW // L):
            acc_v[pl.ds(g * L, L)] = zero
        pltpu.sync_copy(acc_v, out_hbm.at[pl.ds(wid * OUTW, OUTW)])

    return bpr


def kernel(users, items, scores, user_table, item_table):
    B = users.shape[0]
    D = user_table.shape[1]
    bpr = _make_bpr(B, D)
    partials = bpr(users.astype(jnp.int32), items.astype(jnp.int32),
                   scores, user_table.T, item_table.T)
    return jnp.sum(partials) / B
